# baseline (speedup 1.0000x reference)
import jax
import jax.numpy as jnp
from jax import lax
from jax.experimental import pallas as pl
from jax.experimental.pallas import tpu as pltpu
from jax.experimental.pallas import tpu_sc as plsc

B = 4096
F = 26
D = 64
BF = B * F
NC, NS = 2, 16
NW = NC * NS
PER_W = BF // NW
CHUNK = 256
NCH = PER_W // CHUNK
NBUF = 2
LANES = 16
FIELD_SIZE = 100000


def _body(x_hbm, w_hbm, out_hbm, idx_v, pv, buf0, buf1, ob0, ob1,
          gs0, gs1, cs0, cs1):
    wid = lax.axis_index("s") * NC + lax.axis_index("c")
    base = wid * PER_W
    pltpu.sync_copy(x_hbm.at[pl.ds(base, PER_W)], idx_v)

    def off(t, carry):
        pos = base + t * LANES + lax.iota(jnp.int32, LANES)
        sl = pl.ds(t * LANES, LANES)
        v = idx_v[sl] + lax.rem(pos, F) * FIELD_SIZE
        idx_v[sl] = v
        pv[sl] = lax.shift_right_logical(v, 1)
        return carry

    lax.fori_loop(0, PER_W // LANES, off, 0)

    bufs = (buf0, buf1)
    obufs = (ob0, ob1)
    gsems = (gs0, gs1)
    csems = (cs0, cs1)

    def fire_gather(j, b):
        return pltpu.async_copy(
            w_hbm.at[pv.at[pl.ds(j * CHUNK, CHUNK)]], bufs[b], gsems[b])

    def select(j, b):
        def grp(g, carry):
            hv = idx_v[pl.ds(j * CHUNK + g * LANES, LANES)]
            for l in range(LANES):
                h64 = (hv[l] & 1) * D
                srow = g * LANES + l
                orow = g * (LANES // 2) + (l >> 1)
                ocol = (l & 1) * D
                for t in range(D // LANES):
                    obufs[b][orow, pl.ds(ocol + t * LANES, LANES)] = (
                        bufs[b][srow, pl.ds(h64 + t * LANES, LANES)])
            return carry

        lax.fori_loop(0, CHUNK // LANES, grp, 0)

    gathers = [fire_gather(b, b) for b in range(NBUF)]
    copies = [None] * NBUF
    for j in range(NCH):
        b = j % NBUF
        gathers[b].wait()
        if copies[b] is not None:
            copies[b].wait()
        select(j, b)
        copies[b] = pltpu.async_copy(
            obufs[b],
            out_hbm.at[pl.ds(wid * (PER_W // 2) + j * (CHUNK // 2),
                             CHUNK // 2)],
            csems[b])
        nj = j + NBUF
        if nj < NCH:
            gathers[b] = fire_gather(nj, b)
    for j in range(max(0, NCH - NBUF), NCH):
        copies[j % NBUF].wait()


def kernel(x, weight):
    mesh = plsc.VectorSubcoreMesh(
        core_axis_name="c", subcore_axis_name="s",
        num_cores=NC, num_subcores=NS,
    )
    lookup = pl.kernel(
        _body,
        out_type=jax.ShapeDtypeStruct((BF // 2, 2 * D), jnp.float32),
        mesh=mesh,
        scratch_types=[
            pltpu.VMEM((PER_W,), jnp.int32),
            pltpu.VMEM((PER_W,), jnp.int32),
            pltpu.VMEM((CHUNK, 2 * D), jnp.float32),
            pltpu.VMEM((CHUNK, 2 * D), jnp.float32),
            pltpu.VMEM((CHUNK // 2, 2 * D), jnp.float32),
            pltpu.VMEM((CHUNK // 2, 2 * D), jnp.float32),
            pltpu.SemaphoreType.DMA,
            pltpu.SemaphoreType.DMA,
            pltpu.SemaphoreType.DMA,
            pltpu.SemaphoreType.DMA,
        ],
        compiler_params=pltpu.CompilerParams(use_tc_tiling_on_sc=False),
    )
    out = lookup(x.reshape(BF), weight.reshape(weight.shape[0] // 2, 2 * D))
    return out.reshape(B, F * D)


# speedup vs baseline: 1.0000x; 1.0000x over previous
import jax
import jax.numpy as jnp
from jax import lax
from jax.experimental import pallas as pl
from jax.experimental.pallas import tpu as pltpu
from jax.experimental.pallas import tpu_sc as plsc

B = 4096
F = 26
D = 64
BF = B * F
NC, NS = 2, 16
NW = NC * NS
PER_W = BF // NW
CHUNK = 128
NCH = PER_W // CHUNK
NBUF = 2
LANES = 16
FIELD_SIZE = 100000


def _body(x_hbm, w_hbm, out_hbm, idx_v, pv, buf0, buf1, ob0, ob1,
          gs0, gs1, cs0, cs1):
    wid = lax.axis_index("s") * NC + lax.axis_index("c")
    base = wid * PER_W
    pltpu.sync_copy(x_hbm.at[pl.ds(base, PER_W)], idx_v)

    def off(t, carry):
        pos = base + t * LANES + lax.iota(jnp.int32, LANES)
        sl = pl.ds(t * LANES, LANES)
        v = idx_v[sl] + lax.rem(pos, F) * FIELD_SIZE
        idx_v[sl] = v
        pv[sl] = lax.shift_right_logical(v, 1)
        return carry

    lax.fori_loop(0, PER_W // LANES, off, 0)

    bufs = (buf0, buf1)
    obufs = (ob0, ob1)
    gsems = (gs0, gs1)
    csems = (cs0, cs1)

    def fire_gather(j, b):
        return pltpu.async_copy(
            w_hbm.at[pv.at[pl.ds(j * CHUNK, CHUNK)]], bufs[b], gsems[b])

    def select(j, b):
        def grp(g, carry):
            hv = idx_v[pl.ds(j * CHUNK + g * LANES, LANES)]
            for l in range(LANES):
                h64 = (hv[l] & 1) * D
                srow = g * LANES + l
                orow = g * (LANES // 2) + (l >> 1)
                ocol = (l & 1) * D
                for t in range(D // LANES):
                    obufs[b][orow, pl.ds(ocol + t * LANES, LANES)] = (
                        bufs[b][srow, pl.ds(h64 + t * LANES, LANES)])
            return carry

        lax.fori_loop(0, CHUNK // LANES, grp, 0)

    gathers = [fire_gather(b, b) for b in range(NBUF)]
    copies = [None] * NBUF
    for j in range(NCH):
        b = j % NBUF
        gathers[b].wait()
        if copies[b] is not None:
            copies[b].wait()
        select(j, b)
        copies[b] = pltpu.async_copy(
            obufs[b],
            out_hbm.at[pl.ds(wid * (PER_W // 2) + j * (CHUNK // 2),
                             CHUNK // 2)],
            csems[b])
        nj = j + NBUF
        if nj < NCH:
            gathers[b] = fire_gather(nj, b)
    for j in range(max(0, NCH - NBUF), NCH):
        copies[j % NBUF].wait()


def kernel(x, weight):
    mesh = plsc.VectorSubcoreMesh(
        core_axis_name="c", subcore_axis_name="s",
        num_cores=NC, num_subcores=NS,
    )
    lookup = pl.kernel(
        _body,
        out_type=jax.ShapeDtypeStruct((BF // 2, 2 * D), jnp.float32),
        mesh=mesh,
        scratch_types=[
            pltpu.VMEM((PER_W,), jnp.int32),
            pltpu.VMEM((PER_W,), jnp.int32),
            pltpu.VMEM((CHUNK, 2 * D), jnp.float32),
            pltpu.VMEM((CHUNK, 2 * D), jnp.float32),
            pltpu.VMEM((CHUNK // 2, 2 * D), jnp.float32),
            pltpu.VMEM((CHUNK // 2, 2 * D), jnp.float32),
            pltpu.SemaphoreType.DMA,
            pltpu.SemaphoreType.DMA,
            pltpu.SemaphoreType.DMA,
            pltpu.SemaphoreType.DMA,
        ],
    )
    out = lookup(x.reshape(BF), weight.reshape(weight.shape[0] // 2, 2 * D))
    return out.reshape(B, F * D)


# speedup vs baseline: 2.0863x; 2.0862x over previous
import jax
import jax.numpy as jnp
from jax import lax
from jax.experimental import pallas as pl
from jax.experimental.pallas import tpu as pltpu
from jax.experimental.pallas import tpu_sc as plsc

B = 4096
F = 26
D = 64
BF = B * F
NC, NS = 2, 16
NW = NC * NS
PER_W = BF // NW
CHUNK = 32
NCH = PER_W // CHUNK
NROUNDS = NCH // 2
LANES = 16
FIELD_SIZE = 100000


def _body(x_hbm, w_hbm, out_hbm, idx_v, tv, buf0, buf1, ob0, ob1,
          gs0, gs1, cs0, cs1):
    wid = lax.axis_index("s") * NC + lax.axis_index("c")
    base = wid * PER_W
    pltpu.sync_copy(x_hbm.at[pl.ds(base, PER_W)], idx_v)

    def off(t, carry):
        pos = base + t * LANES + lax.iota(jnp.int32, LANES)
        sl = pl.ds(t * LANES, LANES)
        v = idx_v[sl] + lax.rem(pos, F) * FIELD_SIZE
        idx_v[sl] = v
        tv[sl] = lax.shift_right_logical(v, 3)
        return carry

    lax.fori_loop(0, PER_W // LANES, off, 0)

    bufs = (buf0, buf1)
    obufs = (ob0, ob1)
    gsems = (gs0, gs1)
    csems = (cs0, cs1)

    def fire_chunk(j, b):
        for g in range(CHUNK // LANES):
            vec = tv[pl.ds(j * CHUNK + g * LANES, LANES)]
            for l in range(LANES):
                tile = vec[l]
                pltpu.async_copy(
                    w_hbm.at[pl.ds(tile, 1)],
                    bufs[b].at[pl.ds(g * LANES + l, 1)], gsems[b])

    def gather_drain(b):
        pltpu.make_async_copy(
            w_hbm.at[pl.ds(0, CHUNK)], bufs[b], gsems[b]).wait()

    def copy_desc(j, b):
        return pltpu.make_async_copy(
            obufs[b],
            out_hbm.at[pl.ds(wid * (PER_W // 2) + j * (CHUNK // 2),
                             CHUNK // 2)],
            csems[b])

    def select(j, b):
        for g in range(CHUNK // LANES):
            hv = idx_v[pl.ds(j * CHUNK + g * LANES, LANES)]
            for l in range(LANES):
                rit = hv[l] & 7
                srow = g * LANES + l
                orow = g * (LANES // 2) + (l >> 1)
                ocol = (l & 1) * D
                for t in range(D // LANES):
                    obufs[b][orow, pl.ds(ocol + t * LANES, LANES)] = (
                        bufs[b][srow, rit, pl.ds(t * LANES, LANES)])

    fire_chunk(0, 0)
    fire_chunk(1, 1)

    def rnd(k, carry):
        for b in range(2):
            j = 2 * k + b
            gather_drain(b)

            @pl.when(k > 0)
            def _():
                copy_desc(j - 2, b).wait()

            select(j, b)
            copy_desc(j, b).start()

            @pl.when(k < NROUNDS - 1)
            def _():
                fire_chunk(j + 2, b)

        return carry

    lax.fori_loop(0, NROUNDS, rnd, 0)
    copy_desc(NCH - 2, 0).wait()
    copy_desc(NCH - 1, 1).wait()


def kernel(x, weight):
    mesh = plsc.VectorSubcoreMesh(
        core_axis_name="c", subcore_axis_name="s",
        num_cores=NC, num_subcores=NS,
    )
    lookup = pl.kernel(
        _body,
        out_type=jax.ShapeDtypeStruct((BF // 2, 2 * D), jnp.float32),
        mesh=mesh,
        scratch_types=[
            pltpu.VMEM((PER_W,), jnp.int32),
            pltpu.VMEM((PER_W,), jnp.int32),
            pltpu.VMEM((CHUNK, 8, D), jnp.float32),
            pltpu.VMEM((CHUNK, 8, D), jnp.float32),
            pltpu.VMEM((CHUNK // 2, 2 * D), jnp.float32),
            pltpu.VMEM((CHUNK // 2, 2 * D), jnp.float32),
            pltpu.SemaphoreType.DMA,
            pltpu.SemaphoreType.DMA,
            pltpu.SemaphoreType.DMA,
            pltpu.SemaphoreType.DMA,
        ],
    )
    out = lookup(x.reshape(BF), weight.reshape(weight.shape[0] // 8, 8, D))
    return out.reshape(B, F * D)
